# baseline (speedup 1.0000x reference)
import jax
import jax.numpy as jnp
from jax.experimental import pallas as pl
from jax.experimental.pallas import tpu as pltpu
from jax.experimental.pallas import tpu_sc as plsc

BATCH = 16384
NF = 64


_NC = 2
_NS = 16
_NW = _NC * _NS
_CHUNK = 128


def _sc_gather_pair(u_idx2, i_idx2, u_tab2, i_tab2):
    mesh = plsc.VectorSubcoreMesh(core_axis_name="c", subcore_axis_name="s")
    n = u_idx2.shape[0]
    b_per_w = n // _NW
    n_chunks = b_per_w // _CHUNK
    out_type = (
        jax.ShapeDtypeStruct((n, 2 * NF), jnp.float32),
        jax.ShapeDtypeStruct((n, 2 * NF), jnp.float32),
    )

    @pl.kernel(
        out_type=out_type,
        mesh=mesh,
        scratch_types=[
            pltpu.VMEM((b_per_w,), jnp.int32),
            pltpu.VMEM((b_per_w, 2 * NF), jnp.float32),
            pltpu.SemaphoreType.DMA,
        ],
    )
    def gather_kernel(u_idx_hbm, i_idx_hbm, u_tab_hbm, i_tab_hbm,
                      u_out_hbm, i_out_hbm,
                      idx_v, rows_v, sem):
        wid = jax.lax.axis_index("s") * _NC + jax.lax.axis_index("c")
        base = wid * b_per_w

        def one_table(idx_hbm, tab_hbm, out_hbm):
            pltpu.sync_copy(idx_hbm.at[pl.ds(base, b_per_w)], idx_v)
            for j in range(n_chunks):
                sl = pl.ds(j * _CHUNK, _CHUNK)
                pltpu.async_copy(tab_hbm.at[idx_v.at[sl]], rows_v.at[sl], sem)
            for j in range(n_chunks):
                sl = pl.ds(j * _CHUNK, _CHUNK)
                pltpu.make_async_copy(tab_hbm.at[idx_v.at[sl]],
                                      rows_v.at[sl], sem).wait()
            pltpu.sync_copy(rows_v, out_hbm.at[pl.ds(base, b_per_w)])

        one_table(u_idx_hbm, u_tab_hbm, u_out_hbm)
        one_table(i_idx_hbm, i_tab_hbm, i_out_hbm)

    return gather_kernel(u_idx2, i_idx2, u_tab2, i_tab2)


_BT = 2048


def _mlp_body(u2_ref, i2_ref, up_ref, ip_ref, w1u_ref, w1i_ref, b1_ref,
              w2_ref, b2_ref, w3_ref, b3_ref, out_ref):
    u = jnp.where(up_ref[...] > 0, u2_ref[:, NF:], u2_ref[:, :NF])
    i = jnp.where(ip_ref[...] > 0, i2_ref[:, NF:], i2_ref[:, :NF])
    h = jnp.dot(u, w1u_ref[...], preferred_element_type=jnp.float32)
    h += jnp.dot(i, w1i_ref[...], preferred_element_type=jnp.float32)
    h = jnp.maximum(h + b1_ref[...], 0.0)
    h = jnp.dot(h, w2_ref[...], preferred_element_type=jnp.float32)
    h = jnp.maximum(h + b2_ref[...], 0.0)
    out_ref[...] = (
        jnp.dot(h, w3_ref[...], preferred_element_type=jnp.float32)
        + b3_ref[...]
    )


def _tc_mlp(u2, i2, u_par, i_par, W1, b1, W2, b2, W3, b3):
    n = u2.shape[0]
    w1u = W1[:NF]
    w1i = W1[NF:]
    grid = (n // _BT,)
    full = lambda *shape: pl.BlockSpec(shape, lambda g: (0,) * len(shape))
    out = pl.pallas_call(
        _mlp_body,
        grid=grid,
        in_specs=[
            pl.BlockSpec((_BT, 2 * NF), lambda g: (g, 0)),
            pl.BlockSpec((_BT, 2 * NF), lambda g: (g, 0)),
            pl.BlockSpec((_BT, 1), lambda g: (g, 0)),
            pl.BlockSpec((_BT, 1), lambda g: (g, 0)),
            full(NF, W1.shape[1]),
            full(NF, W1.shape[1]),
            full(1, b1.shape[0]),
            full(W2.shape[0], W2.shape[1]),
            full(1, b2.shape[0]),
            full(W3.shape[0], W3.shape[1]),
            full(1, 1),
        ],
        out_specs=pl.BlockSpec((_BT, 1), lambda g: (g, 0)),
        out_shape=jax.ShapeDtypeStruct((n, 1), jnp.float32),
    )(u2, i2, u_par, i_par, w1u, w1i, b1.reshape(1, -1), W2,
      b2.reshape(1, -1), W3, b3.reshape(1, 1))
    return out.reshape(n)


def kernel(users, items, user_table, item_table, W1, b1, W2, b2, W3, b3):
    u_tab2 = user_table.reshape(user_table.shape[0] // 2, 2 * NF)
    i_tab2 = item_table.reshape(item_table.shape[0] // 2, 2 * NF)
    users = users.astype(jnp.int32)
    items = items.astype(jnp.int32)
    u2, i2 = _sc_gather_pair(users // 2, items // 2, u_tab2, i_tab2)
    u_par = (users & 1).reshape(-1, 1)
    i_par = (items & 1).reshape(-1, 1)
    return _tc_mlp(u2, i2, u_par, i_par, W1, b1, W2, b2, W3, b3)
